# Initial kernel scaffold; baseline (speedup 1.0000x reference)
import functools

import jax
import jax.numpy as jnp
from jax import lax
from jax.experimental import pallas as pl
from jax.experimental.pallas import tpu as pltpu
from jax.experimental.pallas import tpu_sc as plsc

DIM = 16
VOCAB_PAD = 48


def _lut_body(table_ref, gamma_ref, beta_ref, out_ref):
    emb = table_ref[...] + lax.broadcasted_iota(jnp.float32, (1, DIM), 1)
    mean = jnp.mean(emb, axis=-1, keepdims=True)
    var = jnp.mean((emb - mean) * (emb - mean), axis=-1, keepdims=True)
    normed = (emb - mean) * lax.rsqrt(var + 1e-5)
    out_ref[...] = jax.nn.sigmoid(normed * gamma_ref[...] + beta_ref[...])


def _compute_lut(emb_table, gamma, beta):
    v = emb_table.shape[0]
    table_p = jnp.pad(emb_table, ((0, VOCAB_PAD - v), (0, 0)))
    return pl.pallas_call(
        _lut_body,
        out_shape=jax.ShapeDtypeStruct((VOCAB_PAD, DIM), jnp.float32),
    )(table_p, gamma.reshape(1, DIM), beta.reshape(1, DIM))


NC, NS = 2, 16
NW = NC * NS
CHUNK = 2048


def _make_gather(total_rows):
    b_per_w = total_rows // NW
    n_chunks = b_per_w // CHUNK
    mesh = plsc.VectorSubcoreMesh(core_axis_name="c", subcore_axis_name="s")

    @functools.partial(
        pl.kernel,
        mesh=mesh,
        out_type=jax.ShapeDtypeStruct((total_rows, DIM), jnp.float32),
        scratch_types=[
            pltpu.VMEM((CHUNK,), jnp.int32),
            pltpu.VMEM((CHUNK, DIM), jnp.float32),
            pltpu.SemaphoreType.DMA,
        ],
    )
    def gather(lut_hbm, idx_hbm, out_hbm, idx_v, rows_v, sem):
        wid = lax.axis_index("s") * NC + lax.axis_index("c")
        base = wid * b_per_w

        def body(g, carry):
            off = base + g * CHUNK
            pltpu.sync_copy(idx_hbm.at[pl.ds(off, CHUNK)], idx_v)
            pltpu.async_copy(lut_hbm.at[idx_v], rows_v, sem).wait()
            pltpu.sync_copy(rows_v, out_hbm.at[pl.ds(off, CHUNK)])
            return carry

        lax.fori_loop(0, n_chunks, body, 0)

    return gather


def kernel(x, emb_table, gamma, beta):
    b, l = x.shape
    lut = _compute_lut(emb_table, gamma, beta)
    idx = x.reshape(-1).astype(jnp.int32)
    out = _make_gather(b * l)(lut, idx)
    return out.reshape(b, l, DIM)


# speedup vs baseline: 2.7505x; 2.7505x over previous
import functools

import jax
import jax.numpy as jnp
from jax import lax
from jax.experimental import pallas as pl
from jax.experimental.pallas import tpu as pltpu
from jax.experimental.pallas import tpu_sc as plsc

DIM = 16
VOCAB_PAD = 48


def _lut_body(table_ref, pos_ref, gamma_ref, beta_ref, out_ref):
    emb = table_ref[...] + pos_ref[...]
    mean = jnp.mean(emb, axis=-1, keepdims=True)
    var = jnp.mean((emb - mean) * (emb - mean), axis=-1, keepdims=True)
    normed = (emb - mean) * lax.rsqrt(var + 1e-5)
    out_ref[...] = jax.nn.sigmoid(normed * gamma_ref[...] + beta_ref[...])


def _compute_lut(emb_table, gamma, beta):
    v = emb_table.shape[0]
    table_p = jnp.pad(emb_table, ((0, VOCAB_PAD - v), (0, 0)))
    pos = jnp.arange(DIM, dtype=jnp.float32).reshape(1, DIM)
    return pl.pallas_call(
        _lut_body,
        out_shape=jax.ShapeDtypeStruct((VOCAB_PAD, DIM), jnp.float32),
    )(table_p, pos, gamma.reshape(1, DIM), beta.reshape(1, DIM))


NC, NS = 2, 16
NW = NC * NS
CHUNK = 2048


def _make_gather(total_rows):
    b_per_w = total_rows // NW
    n_chunks = b_per_w // CHUNK
    mesh = plsc.VectorSubcoreMesh(core_axis_name="c", subcore_axis_name="s")

    @functools.partial(
        pl.kernel,
        mesh=mesh,
        out_type=jax.ShapeDtypeStruct((total_rows, DIM), jnp.float32),
        scratch_types=[
            pltpu.VMEM((CHUNK,), jnp.int32),
            pltpu.VMEM((CHUNK, DIM), jnp.float32),
            pltpu.SemaphoreType.DMA,
        ],
        compiler_params=pltpu.CompilerParams(use_tc_tiling_on_sc=False),
    )
    def gather(lut_hbm, idx_hbm, out_hbm, idx_v, rows_v, sem):
        wid = lax.axis_index("s") * NC + lax.axis_index("c")
        base = wid * b_per_w

        def body(g, carry):
            off = base + g * CHUNK
            pltpu.sync_copy(idx_hbm.at[pl.ds(off, CHUNK)], idx_v)
            pltpu.async_copy(lut_hbm.at[idx_v], rows_v, sem).wait()
            pltpu.sync_copy(rows_v, out_hbm.at[pl.ds(off, CHUNK)])
            return carry

        lax.fori_loop(0, n_chunks, body, 0)

    return gather


def kernel(x, emb_table, gamma, beta):
    b, l = x.shape
    lut = _compute_lut(emb_table, gamma, beta)
    idx = x.reshape(-1).astype(jnp.int32)
    out = _make_gather(b * l)(lut, idx)
    return out.reshape(b, l, DIM)


# speedup vs baseline: 6.9660x; 2.5327x over previous
import functools

import jax
import jax.numpy as jnp
from jax import lax
from jax.experimental import pallas as pl
from jax.experimental.pallas import tpu as pltpu
from jax.experimental.pallas import tpu_sc as plsc

DIM = 16
VOCAB_PAD = 48


def _lut_body(table_ref, pos_ref, gamma_ref, beta_ref, out_ref):
    emb = table_ref[...] + pos_ref[...]
    mean = jnp.mean(emb, axis=-1, keepdims=True)
    var = jnp.mean((emb - mean) * (emb - mean), axis=-1, keepdims=True)
    normed = (emb - mean) * lax.rsqrt(var + 1e-5)
    out_ref[...] = jax.nn.sigmoid(normed * gamma_ref[...] + beta_ref[...])


def _compute_lut(emb_table, gamma, beta):
    v = emb_table.shape[0]
    table_p = jnp.pad(emb_table, ((0, VOCAB_PAD - v), (0, 0)))
    pos = jnp.arange(DIM, dtype=jnp.float32).reshape(1, DIM)
    return pl.pallas_call(
        _lut_body,
        out_shape=jax.ShapeDtypeStruct((VOCAB_PAD, DIM), jnp.float32),
    )(table_p, pos, gamma.reshape(1, DIM), beta.reshape(1, DIM))


NC, NS = 2, 16
NW = NC * NS
CHUNK = 3200
NB = 2


def _make_gather(total_rows):
    b_per_w = total_rows // NW
    n_chunks = b_per_w // CHUNK
    mesh = plsc.VectorSubcoreMesh(core_axis_name="c", subcore_axis_name="s")

    @functools.partial(
        pl.kernel,
        mesh=mesh,
        out_type=jax.ShapeDtypeStruct((total_rows, DIM), jnp.float32),
        scratch_types=[
            pltpu.VMEM_SHARED((VOCAB_PAD, DIM), jnp.float32),
            pltpu.VMEM((NB, CHUNK), jnp.int32),
            pltpu.VMEM((NB, CHUNK, DIM), jnp.float32),
        ]
        + [pltpu.SemaphoreType.DMA] * (3 * NB),
        compiler_params=pltpu.CompilerParams(use_tc_tiling_on_sc=False),
    )
    def gather(lut_hbm, idx_hbm, out_hbm, lut_v, idx_v, rows_v, *sems):
        si, sg, so = sems[0:NB], sems[NB : 2 * NB], sems[2 * NB : 3 * NB]
        sid = lax.axis_index("s")
        wid = sid * NC + lax.axis_index("c")
        base = wid * b_per_w

        @pl.when(sid == 0)
        def _stage_lut():
            pltpu.sync_copy(lut_hbm, lut_v)

        plsc.subcore_barrier()

        idx_d, g_d, o_d = {}, {}, {}

        def fire_idx(c):
            b = c % NB
            idx_d[c] = pltpu.async_copy(
                idx_hbm.at[pl.ds(base + c * CHUNK, CHUNK)], idx_v.at[b], si[b]
            )

        def fire_gather(c):
            b = c % NB
            g_d[c] = pltpu.async_copy(lut_v.at[idx_v.at[b]], rows_v.at[b], sg[b])

        def fire_out(c):
            b = c % NB
            o_d[c] = pltpu.async_copy(
                rows_v.at[b], out_hbm.at[pl.ds(base + c * CHUNK, CHUNK)], so[b]
            )

        for c in range(min(NB, n_chunks)):
            fire_idx(c)
        for g in range(n_chunks):
            idx_d[g].wait()
            if g >= NB:
                o_d[g - NB].wait()
            fire_gather(g)
            if g >= 1:
                g_d[g - 1].wait()
                fire_out(g - 1)
                if g - 1 + NB < n_chunks:
                    fire_idx(g - 1 + NB)
        g_d[n_chunks - 1].wait()
        fire_out(n_chunks - 1)
        for c in range(max(0, n_chunks - NB), n_chunks):
            o_d[c].wait()

    return gather


def kernel(x, emb_table, gamma, beta):
    b, l = x.shape
    lut = _compute_lut(emb_table, gamma, beta)
    idx = x.reshape(-1).astype(jnp.int32)
    out = _make_gather(b * l)(lut, idx)
    return out.reshape(b, l, DIM)
